# speedup vs baseline: 1.0125x; 1.0125x over previous
import functools

import jax
import jax.numpy as jnp
from jax import lax
from jax.experimental import pallas as pl
from jax.experimental.pallas import tpu as pltpu

_LN_EPS = 1e-5


def _fused_kernel(x_ref, wt_ref, b_ref, g_ref, beta_ref, o_ref):
    x = x_ref[...]
    y = jnp.dot(x.astype(jnp.bfloat16), wt_ref[...],
                preferred_element_type=jnp.float32)
    z = y + x + b_ref[...]
    d = z.shape[-1]
    inv_d = jnp.float32(1.0 / d)
    mean = jnp.sum(z, axis=-1, keepdims=True) * inv_d
    ex2 = jnp.sum(z * z, axis=-1, keepdims=True) * inv_d
    var = jnp.maximum(ex2 - mean * mean, 0.0)
    rstd = lax.rsqrt(var + _LN_EPS)
    scale = rstd * g_ref[...]
    shift = beta_ref[...] - mean * scale
    o_ref[...] = (z * scale + shift).astype(o_ref.dtype)


@functools.partial(jax.jit, static_argnames=("tm",))
def _forward(x, w, b, gamma, beta, *, tm=512):
    B, S, D = x.shape
    R = B * S
    TM = min(tm, R)
    n_row = pl.cdiv(R, TM)
    R_pad = n_row * TM

    x2 = x.reshape(R, D)
    if R_pad != R:
        x2 = jnp.pad(x2, ((0, R_pad - R), (0, 0)))
    wt = jnp.asarray(w).T.astype(jnp.bfloat16)
    b2 = b.reshape(1, D).astype(jnp.float32)
    g2 = gamma.reshape(1, D).astype(jnp.float32)
    beta2 = beta.reshape(1, D).astype(jnp.float32)

    out2 = pl.pallas_call(
        _fused_kernel,
        out_shape=jax.ShapeDtypeStruct((R_pad, D), x.dtype),
        grid=(n_row,),
        in_specs=[
            pl.BlockSpec((TM, D), lambda i: (i, 0)),
            pl.BlockSpec((D, D), lambda i: (0, 0)),
            pl.BlockSpec((1, D), lambda i: (0, 0)),
            pl.BlockSpec((1, D), lambda i: (0, 0)),
            pl.BlockSpec((1, D), lambda i: (0, 0)),
        ],
        out_specs=pl.BlockSpec((TM, D), lambda i: (i, 0)),
        compiler_params=pltpu.CompilerParams(
            dimension_semantics=("parallel",),
        ),
    )(x2, wt, b2, g2, beta2)
    return out2[:R].reshape(B, S, D)


def kernel(x, w, b, gamma, beta):
    return _forward(x, w, b, gamma, beta, tm=512)


# speedup vs baseline: 1.1444x; 1.1303x over previous
import functools

import jax
import jax.numpy as jnp
from jax import lax
from jax.experimental import pallas as pl
from jax.experimental.pallas import tpu as pltpu

_LN_EPS = 1e-5


def _fused_kernel(x_ref, wt_ref, b_ref, g_ref, beta_ref, o_ref):
    x = x_ref[...]
    y = jnp.dot(x.astype(jnp.bfloat16), wt_ref[...],
                preferred_element_type=jnp.float32)
    z = y + x + b_ref[...]
    d = z.shape[-1]
    inv_d = jnp.float32(1.0 / d)
    mean = jnp.sum(z, axis=-1, keepdims=True) * inv_d
    ex2 = jnp.sum(z * z, axis=-1, keepdims=True) * inv_d
    var = jnp.maximum(ex2 - mean * mean, 0.0)
    rstd = lax.rsqrt(var + _LN_EPS)
    scale = rstd * g_ref[...]
    shift = beta_ref[...] - mean * scale
    o_ref[...] = (z * scale + shift).astype(o_ref.dtype)


@functools.partial(jax.jit, static_argnames=("tm",))
def _forward(x, w, b, gamma, beta, *, tm=512):
    B, S, D = x.shape
    R = B * S
    TM = min(tm, R)
    n_row = pl.cdiv(R, TM)
    R_pad = n_row * TM

    x2 = x.reshape(R, D)
    if R_pad != R:
        x2 = jnp.pad(x2, ((0, R_pad - R), (0, 0)))
    wt = jnp.asarray(w).T.astype(jnp.bfloat16)
    b2 = b.reshape(1, D).astype(jnp.float32)
    g2 = gamma.reshape(1, D).astype(jnp.float32)
    beta2 = beta.reshape(1, D).astype(jnp.float32)

    out2 = pl.pallas_call(
        _fused_kernel,
        out_shape=jax.ShapeDtypeStruct((R_pad, D), x.dtype),
        grid=(n_row,),
        in_specs=[
            pl.BlockSpec((TM, D), lambda i: (i, 0)),
            pl.BlockSpec((D, D), lambda i: (0, 0)),
            pl.BlockSpec((1, D), lambda i: (0, 0)),
            pl.BlockSpec((1, D), lambda i: (0, 0)),
            pl.BlockSpec((1, D), lambda i: (0, 0)),
        ],
        out_specs=pl.BlockSpec((TM, D), lambda i: (i, 0)),
        compiler_params=pltpu.CompilerParams(
            dimension_semantics=("arbitrary",),
        ),
    )(x2, wt, b2, g2, beta2)
    return out2[:R].reshape(B, S, D)


def kernel(x, w, b, gamma, beta):
    return _forward(x, w, b, gamma, beta, tm=1024)
